# Initial kernel scaffold; baseline (speedup 1.0000x reference)
import jax
import jax.numpy as jnp
from jax.experimental import pallas as pl


def kernel(x, edge_index, Wxz0, Wxz1, bxz, Whz0, Whz1, bhz, Wxr0, Wxr1, bxr, Whr0, Whr1, bhr, Wxh0, Wxh1, bxh, Whh0, Whh1, bhh, Wl, bl):
    raise NotImplementedError("write your pallas kernel here")


# speedup vs baseline: 9.4095x; 9.4095x over previous
import functools

import jax
import jax.numpy as jnp
from jax import lax
from jax.experimental import pallas as pl
from jax.experimental.pallas import tpu as pltpu
from jax.experimental.pallas import tpu_sc as plsc

N, D, F, E = 10000, 256, 256, 160000
NC, NS, L = 2, 16, 16
EP = 163840
CH = 128
ROWS_ALL = EP // CH
NCH_A = EP // (NC * NS) // CH
NCH_B = EP // NS // CH
SROWS = 10240
STRIPE = SROWS // NS
QD = 64
NQ = 4
XROWS = N + 16

_mesh = plsc.VectorSubcoreMesh(core_axis_name="c", subcore_axis_name="s")


def _deg_body(row_hbm, col_hbm, rowp_hbm, deg_hbm,
              row_v, col_v, rp_v, ones_v, z_v, deg_sh):
    c = lax.axis_index("c")
    s = lax.axis_index("s")
    w = s * NC + c

    def _fill_ones(i, _):
        ones_v[i, :] = jnp.ones((L,), jnp.float32)
        return 0
    lax.fori_loop(0, CH, _fill_ones, 0)

    def _fill_z(i, _):
        z_v[i, :] = jnp.zeros((L,), jnp.float32)
        return 0
    lax.fori_loop(0, STRIPE, _fill_z, 0)

    pltpu.sync_copy(z_v, deg_sh.at[pl.ds(s * STRIPE, STRIPE)])
    plsc.subcore_barrier()

    base = w * NCH_A
    pltpu.sync_copy(row_hbm.at[pl.ds(base, NCH_A)], row_v)
    pltpu.sync_copy(col_hbm.at[pl.ds(base, NCH_A)], col_v)

    def _remap(i, _):
        j = i // (CH // L)
        g = i % (CH // L)
        r = row_v[j, pl.ds(g * L, L)]
        cc = col_v[j, pl.ds(g * L, L)]
        rp_v[j, pl.ds(g * L, L)] = jnp.where(r == cc, N, r)
        return 0
    lax.fori_loop(0, NCH_A * (CH // L), _remap, 0)

    def _scat(j, _):
        pltpu.sync_copy(ones_v, deg_sh.at[rp_v.at[j]], add=True)
        return 0
    lax.fori_loop(0, NCH_A, _scat, 0)

    pltpu.sync_copy(rp_v, rowp_hbm.at[pl.ds(base, NCH_A)])
    plsc.subcore_barrier()

    pltpu.sync_copy(deg_sh.at[pl.ds(s * STRIPE, STRIPE)],
                    deg_hbm.at[pl.ds(c * SROWS + s * STRIPE, STRIPE)])


_deg_kernel = functools.partial(
    pl.kernel,
    out_type=(
        jax.ShapeDtypeStruct((ROWS_ALL, CH), jnp.int32),
        jax.ShapeDtypeStruct((NC * SROWS, L), jnp.float32),
    ),
    mesh=_mesh,
    compiler_params=pltpu.CompilerParams(use_tc_tiling_on_sc=False),
    scratch_types=[
        pltpu.VMEM((NCH_A, CH), jnp.int32),
        pltpu.VMEM((NCH_A, CH), jnp.int32),
        pltpu.VMEM((NCH_A, CH), jnp.int32),
        pltpu.VMEM((CH, L), jnp.float32),
        pltpu.VMEM((STRIPE, L), jnp.float32),
        pltpu.VMEM_SHARED((SROWS, L), jnp.float32),
    ],
)(_deg_body)


def _scatter_body(rowp_hbm, colp_hbm, xpad_hbm, out_hbm,
                  rp_v, cp_v, gbuf, z_v, S_sh, sem):
    c = lax.axis_index("c")
    s = lax.axis_index("s")

    def _fill_z(i, _):
        j = i // (QD // L)
        g = i % (QD // L)
        z_v[j, pl.ds(g * L, L)] = jnp.zeros((L,), jnp.float32)
        return 0
    lax.fori_loop(0, (STRIPE // 2) * (QD // L), _fill_z, 0)

    pltpu.sync_copy(colp_hbm.at[pl.ds(s * NCH_B, NCH_B)], cp_v)
    pltpu.sync_copy(rowp_hbm.at[pl.ds(s * NCH_B, NCH_B)], rp_v)

    for q in range(2):
        t = c * 2 + q
        delta = (c * 2) * XROWS if q == 0 else XROWS

        pltpu.sync_copy(z_v, S_sh.at[pl.ds(s * STRIPE, STRIPE // 2)])
        pltpu.sync_copy(z_v,
                        S_sh.at[pl.ds(s * STRIPE + STRIPE // 2, STRIPE // 2)])

        def _addoff(i, _):
            j = i // (CH // L)
            g = i % (CH // L)
            rp_v[j, pl.ds(g * L, L)] = rp_v[j, pl.ds(g * L, L)] + delta
            return 0
        lax.fori_loop(0, NCH_B * (CH // L), _addoff, 0)
        plsc.subcore_barrier()

        def _step(j, _):
            pltpu.async_copy(xpad_hbm.at[rp_v.at[j]], gbuf, sem).wait()
            pltpu.sync_copy(gbuf, S_sh.at[cp_v.at[j]], add=True)
            return 0
        lax.fori_loop(0, NCH_B, _step, 0)

        plsc.subcore_barrier()
        pltpu.sync_copy(S_sh.at[pl.ds(s * STRIPE, STRIPE)],
                        out_hbm.at[pl.ds(t * SROWS + s * STRIPE, STRIPE)])


_scatter_kernel = functools.partial(
    pl.kernel,
    out_type=jax.ShapeDtypeStruct((NQ * SROWS, QD), jnp.float32),
    mesh=_mesh,
    compiler_params=pltpu.CompilerParams(use_tc_tiling_on_sc=False),
    scratch_types=[
        pltpu.VMEM((NCH_B, CH), jnp.int32),
        pltpu.VMEM((NCH_B, CH), jnp.int32),
        pltpu.VMEM((CH, QD), jnp.float32),
        pltpu.VMEM((STRIPE // 2, QD), jnp.float32),
        pltpu.VMEM_SHARED((SROWS, QD), jnp.float32),
        pltpu.SemaphoreType.DMA,
    ],
)(_scatter_body)


def _scale_body(x_ref, dga_ref, dgb_ref, out_ref):
    deg = dga_ref[:, 0:1] + dgb_ref[:, 0:1]
    dinv = jnp.where(deg > 0.0, lax.rsqrt(deg), 0.0)
    out_ref[...] = x_ref[...] * dinv


def _scale_x(x, dga, dgb):
    BR = 1000
    grid = N // BR
    return pl.pallas_call(
        _scale_body,
        grid=(grid,),
        in_specs=[
            pl.BlockSpec((BR, D), lambda i: (i, 0)),
            pl.BlockSpec((BR, L), lambda i: (i, 0)),
            pl.BlockSpec((BR, L), lambda i: (i, 0)),
        ],
        out_specs=pl.BlockSpec((BR, D), lambda i: (i, 0)),
        out_shape=jax.ShapeDtypeStruct((N, D), jnp.float32),
    )(x, dga, dgb)


def _fused_body(x_ref, s_ref, dga_ref, dgb_ref, wz_ref, wh_ref,
                bxz_ref, bhz_ref, bxh_ref, bhh_ref, wl_ref, bl_ref, out_ref):
    deg = dga_ref[:, 0:1] + dgb_ref[:, 0:1]
    ndinv = jnp.where(deg > 0.0, -lax.rsqrt(deg), 0.0)
    u = jnp.concatenate([x_ref[...], ndinv * s_ref[...]], axis=1)
    az = (jnp.dot(u, wz_ref[...], preferred_element_type=jnp.float32)
          + bxz_ref[...] + bhz_ref[...])
    ah = (jnp.dot(u, wh_ref[...], preferred_element_type=jnp.float32)
          + bxh_ref[...] + bhh_ref[...])
    h = jnp.maximum((1.0 - jax.nn.sigmoid(az)) * jnp.tanh(ah), 0.0)
    out_ref[...] = (jnp.sum(h * wl_ref[...], axis=1, keepdims=True)
                    + bl_ref[...])


def _fused_dense(x, s_full, dga, dgb, wz, wh, bxz, bhz, bxh, bhh, wlt, bl2):
    BR = 1000
    grid = N // BR
    row = lambda i: (i, 0)
    full = lambda i: (0, 0)
    return pl.pallas_call(
        _fused_body,
        grid=(grid,),
        in_specs=[
            pl.BlockSpec((BR, D), row),
            pl.BlockSpec((BR, D), row),
            pl.BlockSpec((BR, L), row),
            pl.BlockSpec((BR, L), row),
            pl.BlockSpec((2 * D, F), full),
            pl.BlockSpec((2 * D, F), full),
            pl.BlockSpec((1, F), full),
            pl.BlockSpec((1, F), full),
            pl.BlockSpec((1, F), full),
            pl.BlockSpec((1, F), full),
            pl.BlockSpec((1, F), full),
            pl.BlockSpec((1, 1), full),
        ],
        out_specs=pl.BlockSpec((BR, 1), row),
        out_shape=jax.ShapeDtypeStruct((N, 1), jnp.float32),
    )(x, s_full, dga, dgb, wz, wh, bxz, bhz, bxh, bhh, wlt, bl2)


def kernel(x, edge_index, Wxz0, Wxz1, bxz, Whz0, Whz1, bhz, Wxr0, Wxr1, bxr,
           Whr0, Whr1, bhr, Wxh0, Wxh1, bxh, Whh0, Whh1, bhh, Wl, bl):
    row = edge_index[0]
    col = edge_index[1]
    pad = jnp.zeros((EP - E,), jnp.int32)
    row2 = jnp.concatenate([row, pad]).reshape(ROWS_ALL, CH)
    col2 = jnp.concatenate([col, pad]).reshape(ROWS_ALL, CH)

    rowp2, deg2 = _deg_kernel(row2, col2)
    dga = deg2[:N]
    dgb = deg2[SROWS:SROWS + N]

    xp = _scale_x(x, dga, dgb)
    zrows = jnp.zeros((XROWS - N, QD), jnp.float32)
    xpad = jnp.concatenate(
        [jnp.concatenate([xp[:, t * QD:(t + 1) * QD], zrows], axis=0)
         for t in range(NQ)], axis=0)

    s4 = _scatter_kernel(rowp2, col2, xpad)
    s_full = jnp.concatenate(
        [s4[t * SROWS:t * SROWS + N] for t in range(NQ)], axis=1)

    wz = jnp.concatenate([Wxz0, Wxz1], axis=0)
    wh = jnp.concatenate([Wxh0, Wxh1], axis=0)
    return _fused_dense(x, s_full, dga, dgb, wz, wh,
                        bxz.reshape(1, F), bhz.reshape(1, F),
                        bxh.reshape(1, F), bhh.reshape(1, F),
                        Wl.reshape(1, F), bl.reshape(1, 1))


# speedup vs baseline: 12.5835x; 1.3373x over previous
import functools

import jax
import jax.numpy as jnp
from jax import lax
from jax.experimental import pallas as pl
from jax.experimental.pallas import tpu as pltpu
from jax.experimental.pallas import tpu_sc as plsc

N, D, F, E = 10000, 256, 256, 160000
NC, NS, L = 2, 16, 16
EP = 163840
CH = 128
ROWS_ALL = EP // CH
NCH_A = EP // (NC * NS) // CH
NCH_B = EP // NS // CH
SROWS = 10240
STRIPE = SROWS // NS
HD = 128
NH = 2

_mesh = plsc.VectorSubcoreMesh(core_axis_name="c", subcore_axis_name="s")


def _deg_body(row_hbm, col_hbm, colp_hbm, deg_hbm,
              row_v, col_v, rp_v, ones_v, z_v, deg_sh):
    c = lax.axis_index("c")
    s = lax.axis_index("s")
    w = s * NC + c

    def _fill_ones(i, _):
        ones_v[i, :] = jnp.ones((L,), jnp.float32)
        return 0
    lax.fori_loop(0, CH, _fill_ones, 0)

    def _fill_z(i, _):
        z_v[i, :] = jnp.zeros((L,), jnp.float32)
        return 0
    lax.fori_loop(0, STRIPE, _fill_z, 0)

    pltpu.sync_copy(z_v, deg_sh.at[pl.ds(s * STRIPE, STRIPE)])
    plsc.subcore_barrier()

    base = w * NCH_A
    pltpu.sync_copy(row_hbm.at[pl.ds(base, NCH_A)], row_v)
    pltpu.sync_copy(col_hbm.at[pl.ds(base, NCH_A)], col_v)

    def _remap(i, _):
        j = i // (CH // L)
        g = i % (CH // L)
        r = row_v[j, pl.ds(g * L, L)]
        cc = col_v[j, pl.ds(g * L, L)]
        sl = r == cc
        rp_v[j, pl.ds(g * L, L)] = jnp.where(sl, N, r)
        col_v[j, pl.ds(g * L, L)] = jnp.where(sl, N, cc)
        return 0
    lax.fori_loop(0, NCH_A * (CH // L), _remap, 0)

    def _scat(j, _):
        pltpu.sync_copy(ones_v, deg_sh.at[rp_v.at[j]], add=True)
        return 0
    lax.fori_loop(0, NCH_A, _scat, 0)

    pltpu.sync_copy(col_v, colp_hbm.at[pl.ds(base, NCH_A)])
    plsc.subcore_barrier()

    pltpu.sync_copy(deg_sh.at[pl.ds(s * STRIPE, STRIPE)],
                    deg_hbm.at[pl.ds(c * SROWS + s * STRIPE, STRIPE)])


_deg_kernel = functools.partial(
    pl.kernel,
    out_type=(
        jax.ShapeDtypeStruct((ROWS_ALL, CH), jnp.int32),
        jax.ShapeDtypeStruct((NC * SROWS, L), jnp.float32),
    ),
    mesh=_mesh,
    compiler_params=pltpu.CompilerParams(use_tc_tiling_on_sc=False),
    scratch_types=[
        pltpu.VMEM((NCH_A, CH), jnp.int32),
        pltpu.VMEM((NCH_A, CH), jnp.int32),
        pltpu.VMEM((NCH_A, CH), jnp.int32),
        pltpu.VMEM((CH, L), jnp.float32),
        pltpu.VMEM((STRIPE, L), jnp.float32),
        pltpu.VMEM_SHARED((SROWS, L), jnp.float32),
    ],
)(_deg_body)


HALF = NCH_B // 2
ZR = 16


def _scatter_body(row_hbm, colp_hbm, xtab_hbm, out_hbm,
                  rp_v, cp_v, g0, g1, z_v, S_sh, sem0, sem1):
    c = lax.axis_index("c")
    s = lax.axis_index("s")

    def _fill_z(i, _):
        j = i // (HD // L)
        g = i % (HD // L)
        z_v[j, pl.ds(g * L, L)] = jnp.zeros((L,), jnp.float32)
        return 0
    lax.fori_loop(0, ZR * (HD // L), _fill_z, 0)

    def _zero(i, _):
        pltpu.sync_copy(z_v, S_sh.at[pl.ds(s * STRIPE + i * ZR, ZR)])
        return 0
    lax.fori_loop(0, STRIPE // ZR, _zero, 0)

    delta = c * N

    for half in range(2):
        base = s * NCH_B + half * HALF
        pltpu.sync_copy(colp_hbm.at[pl.ds(base, HALF)], cp_v)
        pltpu.sync_copy(row_hbm.at[pl.ds(base, HALF)], rp_v)

        def _addoff(i, _):
            j = i // (CH // L)
            g = i % (CH // L)
            rp_v[j, pl.ds(g * L, L)] = rp_v[j, pl.ds(g * L, L)] + delta
            return 0
        lax.fori_loop(0, HALF * (CH // L), _addoff, 0)
        if half == 0:
            plsc.subcore_barrier()

        pltpu.async_copy(xtab_hbm.at[rp_v.at[0]], g0, sem0)
        pltpu.async_copy(xtab_hbm.at[rp_v.at[1]], g1, sem1)

        def _pipe(h, _):
            j = 2 * h
            pltpu.make_async_copy(xtab_hbm.at[rp_v.at[j]], g0, sem0).wait()
            pltpu.sync_copy(g0, S_sh.at[cp_v.at[j]], add=True)
            pltpu.async_copy(xtab_hbm.at[rp_v.at[j + 2]], g0, sem0)
            pltpu.make_async_copy(
                xtab_hbm.at[rp_v.at[j + 1]], g1, sem1).wait()
            pltpu.sync_copy(g1, S_sh.at[cp_v.at[j + 1]], add=True)
            pltpu.async_copy(xtab_hbm.at[rp_v.at[j + 3]], g1, sem1)
            return 0
        lax.fori_loop(0, HALF // 2 - 1, _pipe, 0)

        pltpu.make_async_copy(
            xtab_hbm.at[rp_v.at[HALF - 2]], g0, sem0).wait()
        pltpu.sync_copy(g0, S_sh.at[cp_v.at[HALF - 2]], add=True)
        pltpu.make_async_copy(
            xtab_hbm.at[rp_v.at[HALF - 1]], g1, sem1).wait()
        pltpu.sync_copy(g1, S_sh.at[cp_v.at[HALF - 1]], add=True)

    plsc.subcore_barrier()
    pltpu.sync_copy(S_sh.at[pl.ds(s * STRIPE, STRIPE)],
                    out_hbm.at[pl.ds(c * SROWS + s * STRIPE, STRIPE)])


_scatter_kernel = functools.partial(
    pl.kernel,
    out_type=jax.ShapeDtypeStruct((NH * SROWS, HD), jnp.float32),
    mesh=_mesh,
    compiler_params=pltpu.CompilerParams(use_tc_tiling_on_sc=False),
    scratch_types=[
        pltpu.VMEM((HALF, CH), jnp.int32),
        pltpu.VMEM((HALF, CH), jnp.int32),
        pltpu.VMEM((CH, HD), jnp.float32),
        pltpu.VMEM((CH, HD), jnp.float32),
        pltpu.VMEM((ZR, HD), jnp.float32),
        pltpu.VMEM_SHARED((SROWS, HD), jnp.float32),
        pltpu.SemaphoreType.DMA,
        pltpu.SemaphoreType.DMA,
    ],
)(_scatter_body)


def _scale_body(x_ref, dga_ref, dgb_ref, out_ref):
    deg = dga_ref[:, 0:1] + dgb_ref[:, 0:1]
    dinv = jnp.where(deg > 0.0, lax.rsqrt(deg), 0.0)
    out_ref[...] = x_ref[...] * dinv


def _scale_x(x, dga, dgb):
    BR = 1000
    grid = N // BR
    return pl.pallas_call(
        _scale_body,
        grid=(NH, grid),
        in_specs=[
            pl.BlockSpec((BR, HD), lambda h, i: (i, h)),
            pl.BlockSpec((BR, L), lambda h, i: (i, 0)),
            pl.BlockSpec((BR, L), lambda h, i: (i, 0)),
        ],
        out_specs=pl.BlockSpec((BR, HD), lambda h, i: (h * grid + i, 0)),
        out_shape=jax.ShapeDtypeStruct((NH * N, HD), jnp.float32),
    )(x, dga, dgb)


def _fused_body(x_ref, sa_ref, sb_ref, dga_ref, dgb_ref, wz_ref, wh_ref,
                bxz_ref, bhz_ref, bxh_ref, bhh_ref, wl_ref, bl_ref, out_ref):
    deg = dga_ref[:, 0:1] + dgb_ref[:, 0:1]
    ndinv = jnp.where(deg > 0.0, -lax.rsqrt(deg), 0.0)
    u = jnp.concatenate(
        [x_ref[...], ndinv * sa_ref[...], ndinv * sb_ref[...]], axis=1)
    az = (jnp.dot(u, wz_ref[...], preferred_element_type=jnp.float32)
          + bxz_ref[...] + bhz_ref[...])
    ah = (jnp.dot(u, wh_ref[...], preferred_element_type=jnp.float32)
          + bxh_ref[...] + bhh_ref[...])
    h = jnp.maximum((1.0 - jax.nn.sigmoid(az)) * jnp.tanh(ah), 0.0)
    out_ref[...] = (jnp.sum(h * wl_ref[...], axis=1, keepdims=True)
                    + bl_ref[...])


def _fused_dense(x, s2, dga, dgb, wz, wh, bxz, bhz, bxh, bhh, wlt, bl2):
    BR = 80
    grid = N // BR
    row = lambda i: (i, 0)
    rowb = lambda i: (SROWS // BR + i, 0)
    full = lambda i: (0, 0)
    return pl.pallas_call(
        _fused_body,
        grid=(grid,),
        in_specs=[
            pl.BlockSpec((BR, D), row),
            pl.BlockSpec((BR, HD), row),
            pl.BlockSpec((BR, HD), rowb),
            pl.BlockSpec((BR, L), row),
            pl.BlockSpec((BR, L), row),
            pl.BlockSpec((2 * D, F), full),
            pl.BlockSpec((2 * D, F), full),
            pl.BlockSpec((1, F), full),
            pl.BlockSpec((1, F), full),
            pl.BlockSpec((1, F), full),
            pl.BlockSpec((1, F), full),
            pl.BlockSpec((1, F), full),
            pl.BlockSpec((1, 1), full),
        ],
        out_specs=pl.BlockSpec((BR, 1), row),
        out_shape=jax.ShapeDtypeStruct((N, 1), jnp.float32),
    )(x, s2, s2, dga, dgb, wz, wh, bxz, bhz, bxh, bhh, wlt, bl2)


def kernel(x, edge_index, Wxz0, Wxz1, bxz, Whz0, Whz1, bhz, Wxr0, Wxr1, bxr,
           Whr0, Whr1, bhr, Wxh0, Wxh1, bxh, Whh0, Whh1, bhh, Wl, bl):
    row = edge_index[0]
    col = edge_index[1]
    pad = jnp.zeros((EP - E,), jnp.int32)
    row2 = jnp.concatenate([row, pad]).reshape(ROWS_ALL, CH)
    col2 = jnp.concatenate([col, pad]).reshape(ROWS_ALL, CH)

    colp2, deg2 = _deg_kernel(row2, col2)
    dga = deg2[:N]
    dgb = deg2[SROWS:SROWS + N]

    xtab = _scale_x(x, dga, dgb)

    s2 = _scatter_kernel(row2, colp2, xtab)

    wz = jnp.concatenate([Wxz0, Wxz1], axis=0)
    wh = jnp.concatenate([Wxh0, Wxh1], axis=0)
    return _fused_dense(x, s2, dga, dgb, wz, wh,
                        bxz.reshape(1, F), bhz.reshape(1, F),
                        bxh.reshape(1, F), bhh.reshape(1, F),
                        Wl.reshape(1, F), bl.reshape(1, 1))


# speedup vs baseline: 14.4220x; 1.1461x over previous
import functools

import jax
import jax.numpy as jnp
from jax import lax
from jax.experimental import pallas as pl
from jax.experimental.pallas import tpu as pltpu
from jax.experimental.pallas import tpu_sc as plsc

N, D, F, E = 10000, 256, 256, 160000
NC, NS, L = 2, 16, 16
EP = 163840
CH = 128
ROWS_ALL = EP // CH
NCH_A = EP // (NC * NS) // CH
NCH_B = EP // NS // CH
SROWS = 10240
STRIPE = SROWS // NS
HD = 128
NH = 2

_mesh = plsc.VectorSubcoreMesh(core_axis_name="c", subcore_axis_name="s")


def _deg_body(row_hbm, col_hbm, colp_hbm, deg_hbm,
              row_v, col_v, rp_v, ones_v, z_v, deg_sh):
    c = lax.axis_index("c")
    s = lax.axis_index("s")
    w = s * NC + c

    def _fill_ones(i, _):
        ones_v[i, :] = jnp.ones((L,), jnp.float32)
        return 0
    lax.fori_loop(0, CH, _fill_ones, 0)

    def _fill_z(i, _):
        z_v[i, :] = jnp.zeros((L,), jnp.float32)
        return 0
    lax.fori_loop(0, STRIPE, _fill_z, 0)

    pltpu.sync_copy(z_v, deg_sh.at[pl.ds(s * STRIPE, STRIPE)])
    plsc.subcore_barrier()

    base = w * NCH_A
    pltpu.sync_copy(row_hbm.at[pl.ds(base, NCH_A)], row_v)
    pltpu.sync_copy(col_hbm.at[pl.ds(base, NCH_A)], col_v)

    def _remap(i, _):
        j = i // (CH // L)
        g = i % (CH // L)
        r = row_v[j, pl.ds(g * L, L)]
        cc = col_v[j, pl.ds(g * L, L)]
        sl = r == cc
        rp_v[j, pl.ds(g * L, L)] = jnp.where(sl, N, r)
        col_v[j, pl.ds(g * L, L)] = jnp.where(sl, N, cc)
        return 0
    lax.fori_loop(0, NCH_A * (CH // L), _remap, 0)

    def _scat(j, _):
        pltpu.sync_copy(ones_v, deg_sh.at[rp_v.at[j]], add=True)
        return 0
    lax.fori_loop(0, NCH_A, _scat, 0)

    pltpu.sync_copy(col_v, colp_hbm.at[pl.ds(base, NCH_A)])
    plsc.subcore_barrier()

    pltpu.sync_copy(deg_sh.at[pl.ds(s * STRIPE, STRIPE)],
                    deg_hbm.at[pl.ds(c * SROWS + s * STRIPE, STRIPE)])


_deg_kernel = functools.partial(
    pl.kernel,
    out_type=(
        jax.ShapeDtypeStruct((ROWS_ALL, CH), jnp.int32),
        jax.ShapeDtypeStruct((NC * SROWS, L), jnp.float32),
    ),
    mesh=_mesh,
    compiler_params=pltpu.CompilerParams(use_tc_tiling_on_sc=False),
    scratch_types=[
        pltpu.VMEM((NCH_A, CH), jnp.int32),
        pltpu.VMEM((NCH_A, CH), jnp.int32),
        pltpu.VMEM((NCH_A, CH), jnp.int32),
        pltpu.VMEM((CH, L), jnp.float32),
        pltpu.VMEM((STRIPE, L), jnp.float32),
        pltpu.VMEM_SHARED((SROWS, L), jnp.float32),
    ],
)(_deg_body)


HALF = NCH_B // 2
ZR = 16


def _scatter_body(row_hbm, colp_hbm, xtab_hbm, out_hbm,
                  rp_v, cp_v, g0, g1, z_v, S_sh, sem0, sem1):
    c = lax.axis_index("c")
    s = lax.axis_index("s")

    def _fill_z(i, _):
        j = i // (HD // L)
        g = i % (HD // L)
        z_v[j, pl.ds(g * L, L)] = jnp.zeros((L,), jnp.float32)
        return 0
    lax.fori_loop(0, ZR * (HD // L), _fill_z, 0)

    def _zero(i, _):
        pltpu.sync_copy(z_v, S_sh.at[pl.ds(s * STRIPE + i * ZR, ZR)])
        return 0
    lax.fori_loop(0, STRIPE // ZR, _zero, 0)

    delta = c * N

    for half in range(2):
        base = s * NCH_B + half * HALF
        pltpu.sync_copy(colp_hbm.at[pl.ds(base, HALF)], cp_v)
        pltpu.sync_copy(row_hbm.at[pl.ds(base, HALF)], rp_v)

        def _addoff(i, _):
            j = i // (CH // L)
            g = i % (CH // L)
            rp_v[j, pl.ds(g * L, L)] = rp_v[j, pl.ds(g * L, L)] + delta
            return 0
        lax.fori_loop(0, HALF * (CH // L), _addoff, 0)
        if half == 0:
            plsc.subcore_barrier()

        pltpu.async_copy(xtab_hbm.at[rp_v.at[0]], g0, sem0)
        pltpu.async_copy(xtab_hbm.at[rp_v.at[1]], g1, sem1)

        def _pipe(h, _):
            j = 2 * h
            pltpu.make_async_copy(xtab_hbm.at[rp_v.at[j]], g0, sem0).wait()
            pltpu.sync_copy(g0, S_sh.at[cp_v.at[j]], add=True)
            pltpu.async_copy(xtab_hbm.at[rp_v.at[j + 2]], g0, sem0)
            pltpu.make_async_copy(
                xtab_hbm.at[rp_v.at[j + 1]], g1, sem1).wait()
            pltpu.sync_copy(g1, S_sh.at[cp_v.at[j + 1]], add=True)
            pltpu.async_copy(xtab_hbm.at[rp_v.at[j + 3]], g1, sem1)
            return 0
        lax.fori_loop(0, HALF // 2 - 1, _pipe, 0)

        pltpu.make_async_copy(
            xtab_hbm.at[rp_v.at[HALF - 2]], g0, sem0).wait()
        pltpu.sync_copy(g0, S_sh.at[cp_v.at[HALF - 2]], add=True)
        pltpu.make_async_copy(
            xtab_hbm.at[rp_v.at[HALF - 1]], g1, sem1).wait()
        pltpu.sync_copy(g1, S_sh.at[cp_v.at[HALF - 1]], add=True)

    plsc.subcore_barrier()
    pltpu.sync_copy(S_sh.at[pl.ds(s * STRIPE, STRIPE)],
                    out_hbm.at[pl.ds(c * SROWS + s * STRIPE, STRIPE)])


_scatter_kernel = functools.partial(
    pl.kernel,
    out_type=jax.ShapeDtypeStruct((NH * SROWS, HD), jnp.float32),
    mesh=_mesh,
    compiler_params=pltpu.CompilerParams(use_tc_tiling_on_sc=False),
    scratch_types=[
        pltpu.VMEM((HALF, CH), jnp.int32),
        pltpu.VMEM((HALF, CH), jnp.int32),
        pltpu.VMEM((CH, HD), jnp.float32),
        pltpu.VMEM((CH, HD), jnp.float32),
        pltpu.VMEM((ZR, HD), jnp.float32),
        pltpu.VMEM_SHARED((SROWS, HD), jnp.float32),
        pltpu.SemaphoreType.DMA,
        pltpu.SemaphoreType.DMA,
    ],
)(_scatter_body)


def _scale_body(x_ref, dga_ref, dgb_ref, out_ref):
    deg = dga_ref[:, 0:1] + dgb_ref[:, 0:1]
    dinv = jnp.where(deg > 0.0, lax.rsqrt(deg), 0.0)
    out_ref[...] = x_ref[...] * dinv


def _scale_x(x, dga, dgb):
    BR = 1000
    grid = N // BR
    return pl.pallas_call(
        _scale_body,
        grid=(NH, grid),
        in_specs=[
            pl.BlockSpec((BR, HD), lambda h, i: (i, h)),
            pl.BlockSpec((BR, L), lambda h, i: (i, 0)),
            pl.BlockSpec((BR, L), lambda h, i: (i, 0)),
        ],
        out_specs=pl.BlockSpec((BR, HD), lambda h, i: (h * grid + i, 0)),
        out_shape=jax.ShapeDtypeStruct((NH * N, HD), jnp.float32),
    )(x, dga, dgb)


def _xw_body(x_ref, w_ref, out_ref):
    out_ref[...] = jnp.dot(x_ref[...], w_ref[...],
                           preferred_element_type=jnp.float32)


def _xw_dense(x, w0):
    BR = 2000
    grid = N // BR
    return pl.pallas_call(
        _xw_body,
        grid=(grid,),
        in_specs=[
            pl.BlockSpec((BR, D), lambda i: (i, 0)),
            pl.BlockSpec((D, 2 * F), lambda i: (0, 0)),
        ],
        out_specs=pl.BlockSpec((BR, 2 * F), lambda i: (i, 0)),
        out_shape=jax.ShapeDtypeStruct((N, 2 * F), jnp.float32),
    )(x, w0)


def _fused_body(p_ref, sa_ref, sb_ref, dga_ref, dgb_ref, w1_ref,
                bz_ref, bh_ref, wl_ref, bl_ref, out_ref):
    deg = dga_ref[:, 0:1] + dgb_ref[:, 0:1]
    ndinv = jnp.where(deg > 0.0, -lax.rsqrt(deg), 0.0)
    t = jnp.concatenate([ndinv * sa_ref[...], ndinv * sb_ref[...]], axis=1)
    tw = jnp.dot(t, w1_ref[...], preferred_element_type=jnp.float32)
    az = p_ref[:, :F] + tw[:, :F] + bz_ref[...]
    ah = p_ref[:, F:] + tw[:, F:] + bh_ref[...]
    h = jnp.maximum((1.0 - jax.nn.sigmoid(az)) * jnp.tanh(ah), 0.0)
    out_ref[...] = (jnp.sum(h * wl_ref[...], axis=1, keepdims=True)
                    + bl_ref[...])


def _fused_dense(p, sa, sb, dga, dgb, w1, bz, bh, wlt, bl2):
    BR = 1000
    grid = N // BR
    row = lambda i: (i, 0)
    full = lambda i: (0, 0)
    return pl.pallas_call(
        _fused_body,
        grid=(grid,),
        in_specs=[
            pl.BlockSpec((BR, 2 * F), row),
            pl.BlockSpec((BR, HD), row),
            pl.BlockSpec((BR, HD), row),
            pl.BlockSpec((BR, L), row),
            pl.BlockSpec((BR, L), row),
            pl.BlockSpec((D, 2 * F), full),
            pl.BlockSpec((1, F), full),
            pl.BlockSpec((1, F), full),
            pl.BlockSpec((1, F), full),
            pl.BlockSpec((1, 1), full),
        ],
        out_specs=pl.BlockSpec((BR, 1), row),
        out_shape=jax.ShapeDtypeStruct((N, 1), jnp.float32),
    )(p, sa, sb, dga, dgb, w1, bz, bh, wlt, bl2)


def kernel(x, edge_index, Wxz0, Wxz1, bxz, Whz0, Whz1, bhz, Wxr0, Wxr1, bxr,
           Whr0, Whr1, bhr, Wxh0, Wxh1, bxh, Whh0, Whh1, bhh, Wl, bl):
    row = edge_index[0]
    col = edge_index[1]
    pad = jnp.zeros((EP - E,), jnp.int32)
    row2 = jnp.concatenate([row, pad]).reshape(ROWS_ALL, CH)
    col2 = jnp.concatenate([col, pad]).reshape(ROWS_ALL, CH)

    colp2, deg2 = _deg_kernel(row2, col2)
    dga = deg2[:N]
    dgb = deg2[SROWS:SROWS + N]

    xtab = _scale_x(x, dga, dgb)

    p = _xw_dense(x, jnp.concatenate([Wxz0, Wxh0], axis=1))

    s2 = _scatter_kernel(row2, colp2, xtab)
    sa = lax.slice(s2, (0, 0), (N, HD))
    sb = lax.slice(s2, (SROWS, 0), (SROWS + N, HD))

    w1 = jnp.concatenate([Wxz1, Wxh1], axis=1)
    return _fused_dense(p, sa, sb, dga, dgb, w1,
                        (bxz + bhz).reshape(1, F), (bxh + bhh).reshape(1, F),
                        Wl.reshape(1, F), bl.reshape(1, 1))
